# baseline (speedup 1.0000x reference)
import jax
import jax.numpy as jnp
from jax.experimental import pallas as pl
from jax.experimental.pallas import tpu as pltpu

_IN_NF = 32
_HID = 64
_OUT_NF = 16
_N_LAYERS = 2
_B = 8
_N = 256
_N2 = _N // 2
_T = 128
_BPG = 2


def _silu(v):
    return v * jax.nn.sigmoid(v)


def _fwd(h_ref, x_ref, *rest):
    out_ref = rest[-1]
    w = [r[...] for r in rest[:-1]]
    it = iter(w)

    emb_in_t, emb_in_b = next(it), next(it)
    layers = []
    for _ in range(_N_LAYERS):
        layers.append(dict(
            wr_t=next(it), wc_t=next(it), const=next(it),
            wradl=next(it), wradr=next(it), constp=next(it),
            we1_t=next(it), be1=next(it), we1_p=next(it), be1_p=next(it),
            wc0_p=next(it), bc0_p=next(it), wrep=next(it),
            wn0h_t=next(it), wn0a_t=next(it), bn0=next(it),
            wn1_t=next(it), bn1=next(it),
        ))
    emb_out_t, emb_out_b = next(it), next(it)
    m0_t, m0_b, m1_t, m1_b = next(it), next(it), next(it), next(it)

    for bi in range(_BPG):
        _one_batch(h_ref[bi], x_ref[bi], out_ref, bi, layers,
                   emb_in_t, emb_in_b, emb_out_t, emb_out_b,
                   m0_t, m0_b, m1_t, m1_b)


def _one_batch(h0, x0, out_ref, bi, layers, emb_in_t, emb_in_b,
               emb_out_t, emb_out_b, m0_t, m0_b, m1_t, m1_b):
    h = jnp.dot(h0, emb_in_t) + emb_in_b
    x = x0

    for lp in layers:
        hr = jnp.dot(h, lp["wr_t"])
        hc = jnp.dot(h, lp["wc_t"])

        ef_d = _silu(jnp.dot(_silu(hr + hc + lp["const"]), lp["we1_t"])
                     + lp["be1"])

        bf = jnp.bfloat16
        hrp = jnp.concatenate([hr, hr], axis=1).astype(bf)
        hcp = (jnp.concatenate([hc[:_N2], hc[_N2:]], axis=1)
               + lp["constp"]).astype(bf)
        wradl3 = lp["wradl"].reshape(1, 1, 2 * _HID).astype(bf)
        wradr3 = lp["wradr"].reshape(1, 1, 2 * _HID).astype(bf)
        we1_pb = lp["we1_p"].astype(bf)
        be1_pb = lp["be1_p"].astype(bf)
        wc0_pb = lp["wc0_p"].astype(bf)
        bc0_pb = lp["bc0_p"].astype(bf)
        wrep_b = lp["wrep"].astype(bf)

        xx = jnp.sum(x * x, axis=1)
        xxr = xx.reshape(1, _N)
        gram = jax.lax.dot_general(x, x, (((1,), (1,)), ((), ())))
        x1, x2 = x[:_N2], x[_N2:]

        on1 = jnp.ones((_N2, 1), dtype=jnp.float32)
        zpad = jnp.zeros((_N2, _HID - 4), dtype=jnp.float32)
        uaug = jnp.concatenate([x1, on1, zpad, x2, on1, zpad],
                               axis=1).reshape(1, _N2, 2 * _HID)

        agg_tiles, num_tiles = [], []
        for t in range(_N // _T):
            sl = slice(t * _T, (t + 1) * _T)
            radial = xx[sl].reshape(_T, 1) + xxr - 2.0 * gram[sl]
            rad1 = radial[:, :_N2].astype(bf)
            rad2 = radial[:, _N2:].astype(bf)
            t0 = ((rad1[:, :, None] * wradl3 + hrp[sl][:, None, :])
                  + (rad2[:, :, None] * wradr3 + hcp[None, :, :]))
            s1 = _silu(t0).reshape(_T * _N2, 2 * _HID)
            z1 = jnp.dot(s1, we1_pb, preferred_element_type=jnp.float32)
            ef = _silu(z1.astype(bf) + be1_pb)
            z2 = jnp.dot(ef, wc0_pb, preferred_element_type=jnp.float32)
            c0 = _silu(z2.astype(bf) + bc0_pb)
            srep = jnp.dot(c0, wrep_b, preferred_element_type=jnp.float32)
            p3 = srep.reshape(_T, _N2, 2 * _HID) * uaug
            r = jnp.sum(p3, axis=1)

            aggp = jnp.sum(ef.reshape(_T, _N2, 2 * _HID), axis=1,
                           dtype=jnp.float32)
            agg_tiles.append(aggp[:, :_HID] + aggp[:, _HID:])
            srow = r[:, 3:4] + r[:, _HID + 3:_HID + 4]
            sx = r[:, 0:3] + r[:, _HID:_HID + 3]
            num_tiles.append(x[sl] * srow - sx)

        agg = jnp.concatenate(agg_tiles, axis=0) - ef_d
        num = jnp.concatenate(num_tiles, axis=0)
        x = x + num * (1.0 / (_N - 1))

        hn = _silu(jnp.dot(h, lp["wn0h_t"]) + jnp.dot(agg, lp["wn0a_t"])
                   + lp["bn0"])
        h = h + jnp.dot(hn, lp["wn1_t"]) + lp["bn1"]

    h = jnp.dot(h, emb_out_t) + emb_out_b
    pool = jnp.sum(h, axis=0, keepdims=True) * (1.0 / _N)
    z = jnp.maximum(jnp.dot(pool, m0_t) + m0_b, 0.0)
    out_ref[bi, :, :] = jnp.dot(z, m1_t) + m1_b


def _blockdiag2(wt):
    z = jnp.zeros_like(wt)
    return jnp.concatenate([jnp.concatenate([wt, z], axis=1),
                            jnp.concatenate([z, wt], axis=1)], axis=0)


def kernel(h, x, params):
    p = params
    zh = jnp.zeros((1, _HID), dtype=jnp.float32)
    z1 = jnp.zeros((_HID, 1), dtype=jnp.float32)
    ws = [p["emb_in"]["W"].T, p["emb_in"]["b"].reshape(1, _HID)]
    for lp in p["layers"]:
        we0 = lp["edge0"]["W"]
        wrad = we0[:, 2 * _HID].reshape(1, _HID)
        const = (lp["edge0"]["b"] + we0[:, 2 * _HID + 1]).reshape(1, _HID)
        we1_t = lp["edge1"]["W"].T
        be1 = lp["edge1"]["b"].reshape(1, _HID)
        wc1_t = lp["coord1"]["W"].T
        ws += [
            we0[:, :_HID].T,
            we0[:, _HID:2 * _HID].T,
            const,
            jnp.concatenate([wrad, zh], axis=1),
            jnp.concatenate([zh, wrad], axis=1),
            jnp.concatenate([const, const], axis=1),
            we1_t, be1,
            _blockdiag2(we1_t),
            jnp.concatenate([be1, be1], axis=1),
            _blockdiag2(lp["coord0"]["W"].T),
            jnp.concatenate([lp["coord0"]["b"].reshape(1, _HID)] * 2, axis=1),
            jnp.concatenate(
                [jnp.tile(jnp.concatenate([wc1_t, z1], axis=0), (1, _HID)),
                 jnp.tile(jnp.concatenate([z1, wc1_t], axis=0), (1, _HID))],
                axis=1),
            lp["node0"]["W"][:, :_HID].T,
            lp["node0"]["W"][:, _HID:].T,
            lp["node0"]["b"].reshape(1, _HID),
            lp["node1"]["W"].T, lp["node1"]["b"].reshape(1, _HID),
        ]
    ws += [p["emb_out"]["W"].T, p["emb_out"]["b"].reshape(1, _HID),
           p["mlp0"]["W"].T, p["mlp0"]["b"].reshape(1, _HID),
           p["mlp1"]["W"].T, p["mlp1"]["b"].reshape(1, _OUT_NF)]

    w_specs = [pl.BlockSpec(a.shape, lambda b: (0,) * a.ndim) for a in ws]
    out = pl.pallas_call(
        _fwd,
        grid=(_B // _BPG,),
        in_specs=[pl.BlockSpec((_BPG, _N, _IN_NF), lambda b: (b, 0, 0)),
                  pl.BlockSpec((_BPG, _N, 3), lambda b: (b, 0, 0))] + w_specs,
        out_specs=pl.BlockSpec((_BPG, 1, _OUT_NF), lambda b: (b, 0, 0)),
        out_shape=jax.ShapeDtypeStruct((_B, 1, _OUT_NF), jnp.float32),
        compiler_params=pltpu.CompilerParams(
            dimension_semantics=("parallel",)),
    )(h, x, *ws)
    return out.reshape(_B, _OUT_NF)


# speedup vs baseline: 1.5794x; 1.5794x over previous
import jax
import jax.numpy as jnp
from jax.experimental import pallas as pl
from jax.experimental.pallas import tpu as pltpu

_IN_NF = 32
_HID = 64
_OUT_NF = 16
_N_LAYERS = 2
_B = 8
_N = 256
_N2 = _N // 2
_T = 256


def _silu(v):
    h = 0.5 * v
    return h * jnp.tanh(h) + h


def _fwd(h_ref, x_ref, *rest):
    out_ref = rest[-1]
    w = [r[...] for r in rest[:-1]]
    it = iter(w)

    emb_in_t, emb_in_b = next(it), next(it)
    layers = []
    for _ in range(_N_LAYERS):
        layers.append(dict(
            wr_t=next(it), wc_t=next(it), const=next(it),
            wradl=next(it), wradr=next(it), constp=next(it),
            we1_t=next(it), be1=next(it), we1_p=next(it), be1_p=next(it),
            wc0_p=next(it), bc0_p=next(it), wrep=next(it),
            wn0h_t=next(it), wn0a_t=next(it), bn0=next(it),
            wn1_t=next(it), bn1=next(it),
        ))
    emb_out_t, emb_out_b = next(it), next(it)
    m0_t, m0_b, m1_t, m1_b = next(it), next(it), next(it), next(it)

    h = jnp.dot(h_ref[0], emb_in_t) + emb_in_b
    x = x_ref[0]

    for lp in layers:
        hr = jnp.dot(h, lp["wr_t"])
        hc = jnp.dot(h, lp["wc_t"])

        ef_d = _silu(jnp.dot(_silu(hr + hc + lp["const"]), lp["we1_t"])
                     + lp["be1"])

        bf = jnp.bfloat16
        hrp = jnp.concatenate([hr, hr], axis=1).astype(bf)
        hcp = (jnp.concatenate([hc[:_N2], hc[_N2:]], axis=1)
               + lp["constp"]).astype(bf)
        wradl3 = lp["wradl"].reshape(1, 1, 2 * _HID).astype(bf)
        wradr3 = lp["wradr"].reshape(1, 1, 2 * _HID).astype(bf)
        we1_pb = lp["we1_p"].astype(bf)
        be1_pb = lp["be1_p"].astype(bf)
        wc0_pb = lp["wc0_p"].astype(bf)
        bc0_pb = lp["bc0_p"].astype(bf)
        wrep_b = lp["wrep"].astype(bf)

        xx = jnp.sum(x * x, axis=1)
        xxr = xx.reshape(1, _N)
        gram = jax.lax.dot_general(x, x, (((1,), (1,)), ((), ())))
        x1, x2 = x[:_N2], x[_N2:]

        on1 = jnp.ones((_N2, 1), dtype=jnp.float32)
        zpad = jnp.zeros((_N2, _HID - 4), dtype=jnp.float32)
        uaug = jnp.concatenate([x1, on1, zpad, x2, on1, zpad],
                               axis=1).reshape(1, _N2, 2 * _HID)

        agg_tiles, num_tiles = [], []
        for t in range(_N // _T):
            sl = slice(t * _T, (t + 1) * _T)
            radial = xx[sl].reshape(_T, 1) + xxr - 2.0 * gram[sl]
            rad1 = radial[:, :_N2].astype(bf)
            rad2 = radial[:, _N2:].astype(bf)
            t0 = ((rad1[:, :, None] * wradl3 + hrp[sl][:, None, :])
                  + (rad2[:, :, None] * wradr3 + hcp[None, :, :]))
            s1 = _silu(t0).reshape(_T * _N2, 2 * _HID)
            z1 = jnp.dot(s1, we1_pb, preferred_element_type=jnp.float32)
            ef = _silu(z1.astype(bf) + be1_pb)
            z2 = jnp.dot(ef, wc0_pb, preferred_element_type=jnp.float32)
            c0 = _silu(z2.astype(bf) + bc0_pb)
            srep = jnp.dot(c0, wrep_b, preferred_element_type=jnp.float32)
            p3 = srep.reshape(_T, _N2, 2 * _HID) * uaug
            r = jnp.sum(p3, axis=1)

            aggp = jnp.sum(ef.reshape(_T, _N2, 2 * _HID), axis=1,
                           dtype=jnp.float32)
            agg_tiles.append(aggp[:, :_HID] + aggp[:, _HID:])
            srow = r[:, 3:4] + r[:, _HID + 3:_HID + 4]
            sx = r[:, 0:3] + r[:, _HID:_HID + 3]
            num_tiles.append(x[sl] * srow - sx)

        agg = jnp.concatenate(agg_tiles, axis=0) - ef_d
        num = jnp.concatenate(num_tiles, axis=0)
        x = x + num * (1.0 / (_N - 1))

        hn = _silu(jnp.dot(h, lp["wn0h_t"]) + jnp.dot(agg, lp["wn0a_t"])
                   + lp["bn0"])
        h = h + jnp.dot(hn, lp["wn1_t"]) + lp["bn1"]

    h = jnp.dot(h, emb_out_t) + emb_out_b
    pool = jnp.sum(h, axis=0, keepdims=True) * (1.0 / _N)
    z = jnp.maximum(jnp.dot(pool, m0_t) + m0_b, 0.0)
    out_ref[...] = (jnp.dot(z, m1_t) + m1_b).reshape(1, 1, _OUT_NF)


def _blockdiag2(wt):
    z = jnp.zeros_like(wt)
    return jnp.concatenate([jnp.concatenate([wt, z], axis=1),
                            jnp.concatenate([z, wt], axis=1)], axis=0)


def kernel(h, x, params):
    p = params
    zh = jnp.zeros((1, _HID), dtype=jnp.float32)
    z1 = jnp.zeros((_HID, 1), dtype=jnp.float32)
    ws = [p["emb_in"]["W"].T, p["emb_in"]["b"].reshape(1, _HID)]
    for lp in p["layers"]:
        we0 = lp["edge0"]["W"]
        wrad = we0[:, 2 * _HID].reshape(1, _HID)
        const = (lp["edge0"]["b"] + we0[:, 2 * _HID + 1]).reshape(1, _HID)
        we1_t = lp["edge1"]["W"].T
        be1 = lp["edge1"]["b"].reshape(1, _HID)
        wc1_t = lp["coord1"]["W"].T
        ws += [
            we0[:, :_HID].T,
            we0[:, _HID:2 * _HID].T,
            const,
            jnp.concatenate([wrad, zh], axis=1),
            jnp.concatenate([zh, wrad], axis=1),
            jnp.concatenate([const, const], axis=1),
            we1_t, be1,
            _blockdiag2(we1_t),
            jnp.concatenate([be1, be1], axis=1),
            _blockdiag2(lp["coord0"]["W"].T),
            jnp.concatenate([lp["coord0"]["b"].reshape(1, _HID)] * 2, axis=1),
            jnp.concatenate(
                [jnp.tile(jnp.concatenate([wc1_t, z1], axis=0), (1, _HID)),
                 jnp.tile(jnp.concatenate([z1, wc1_t], axis=0), (1, _HID))],
                axis=1),
            lp["node0"]["W"][:, :_HID].T,
            lp["node0"]["W"][:, _HID:].T,
            lp["node0"]["b"].reshape(1, _HID),
            lp["node1"]["W"].T, lp["node1"]["b"].reshape(1, _HID),
        ]
    ws += [p["emb_out"]["W"].T, p["emb_out"]["b"].reshape(1, _HID),
           p["mlp0"]["W"].T, p["mlp0"]["b"].reshape(1, _HID),
           p["mlp1"]["W"].T, p["mlp1"]["b"].reshape(1, _OUT_NF)]

    w_specs = [pl.BlockSpec(a.shape, lambda b: (0,) * a.ndim) for a in ws]
    out = pl.pallas_call(
        _fwd,
        grid=(_B,),
        in_specs=[pl.BlockSpec((1, _N, _IN_NF), lambda b: (b, 0, 0)),
                  pl.BlockSpec((1, _N, 3), lambda b: (b, 0, 0))] + w_specs,
        out_specs=pl.BlockSpec((1, 1, _OUT_NF), lambda b: (b, 0, 0)),
        out_shape=jax.ShapeDtypeStruct((_B, 1, _OUT_NF), jnp.float32),
        compiler_params=pltpu.CompilerParams(
            dimension_semantics=("parallel",)),
    )(h, x, *ws)
    return out.reshape(_B, _OUT_NF)


# speedup vs baseline: 1.6820x; 1.0650x over previous
import jax
import jax.numpy as jnp
from jax.experimental import pallas as pl
from jax.experimental.pallas import tpu as pltpu

_IN_NF = 32
_HID = 64
_OUT_NF = 16
_N_LAYERS = 2
_B = 8
_N = 256
_N2 = _N // 2
_T = 128


def _silu(v):
    h = 0.5 * v
    return h * jnp.tanh(h) + h


def _fwd(h_ref, x_ref, *rest):
    out_ref = rest[-1]
    w = [r[...] for r in rest[:-1]]
    it = iter(w)

    emb_in_t, emb_in_b = next(it), next(it)
    layers = []
    for _ in range(_N_LAYERS):
        layers.append(dict(
            wr_t=next(it), wc_t=next(it), const=next(it),
            wradl=next(it), wradr=next(it), constp=next(it),
            we1_t=next(it), be1=next(it), we1_p=next(it), be1_p=next(it),
            wc0_p=next(it), bc0_p=next(it), wrep=next(it),
            wn0h_t=next(it), wn0a_t=next(it), bn0=next(it),
            wn1_t=next(it), bn1=next(it),
        ))
    emb_out_t, emb_out_b = next(it), next(it)
    m0_t, m0_b, m1_t, m1_b = next(it), next(it), next(it), next(it)

    h = jnp.dot(h_ref[0], emb_in_t) + emb_in_b
    x = x_ref[0]

    for lp in layers:
        hr = jnp.dot(h, lp["wr_t"])
        hc = jnp.dot(h, lp["wc_t"])

        ef_d = _silu(jnp.dot(_silu(hr + hc + lp["const"]), lp["we1_t"])
                     + lp["be1"])

        bf = jnp.bfloat16
        hrp = jnp.concatenate([hr, hr], axis=1).astype(bf)
        hcp = (jnp.concatenate([hc[:_N2], hc[_N2:]], axis=1)
               + lp["constp"]).astype(bf)
        wradl3 = lp["wradl"].reshape(1, 1, 2 * _HID).astype(bf)
        wradr3 = lp["wradr"].reshape(1, 1, 2 * _HID).astype(bf)
        we1_pb = lp["we1_p"].astype(bf)
        be1_pb = lp["be1_p"].astype(bf)
        wc0_pb = lp["wc0_p"].astype(bf)
        bc0_pb = lp["bc0_p"].astype(bf)
        wrep_b = lp["wrep"].astype(bf)

        xx = jnp.sum(x * x, axis=1)
        xxr = xx.reshape(1, _N)
        gram = jax.lax.dot_general(x, x, (((1,), (1,)), ((), ())))
        x1, x2 = x[:_N2], x[_N2:]

        on1 = jnp.ones((_N2, 1), dtype=jnp.float32)
        zpad = jnp.zeros((_N2, _HID - 4), dtype=jnp.float32)
        uaug = jnp.concatenate([x1, on1, zpad, x2, on1, zpad],
                               axis=1).reshape(1, _N2, 2 * _HID)

        agg_tiles, num_tiles = [], []
        for t in range(_N // _T):
            sl = slice(t * _T, (t + 1) * _T)
            radial = xx[sl].reshape(_T, 1) + xxr - 2.0 * gram[sl]
            rad1 = radial[:, :_N2].astype(bf)
            rad2 = radial[:, _N2:].astype(bf)
            t0 = ((rad1[:, :, None] * wradl3 + hrp[sl][:, None, :])
                  + (rad2[:, :, None] * wradr3 + hcp[None, :, :]))
            s1 = _silu(t0).reshape(_T * _N2, 2 * _HID)
            z1 = jnp.dot(s1, we1_pb, preferred_element_type=jnp.float32)
            ef = _silu(z1.astype(bf) + be1_pb)
            z2 = jnp.dot(ef, wc0_pb, preferred_element_type=jnp.float32)
            c0 = _silu(z2.astype(bf) + bc0_pb)
            srep = jnp.dot(c0, wrep_b, preferred_element_type=jnp.float32)
            p3 = srep.reshape(_T, _N2, 2 * _HID) * uaug
            r = jnp.sum(p3, axis=1)

            aggp = jnp.sum(ef.reshape(_T, _N2, 2 * _HID), axis=1,
                           dtype=jnp.float32)
            agg_tiles.append(aggp[:, :_HID] + aggp[:, _HID:])
            srow = r[:, 3:4] + r[:, _HID + 3:_HID + 4]
            sx = r[:, 0:3] + r[:, _HID:_HID + 3]
            num_tiles.append(x[sl] * srow - sx)

        agg = jnp.concatenate(agg_tiles, axis=0) - ef_d
        num = jnp.concatenate(num_tiles, axis=0)
        x = x + num * (1.0 / (_N - 1))

        hn = _silu(jnp.dot(h, lp["wn0h_t"]) + jnp.dot(agg, lp["wn0a_t"])
                   + lp["bn0"])
        h = h + jnp.dot(hn, lp["wn1_t"]) + lp["bn1"]

    h = jnp.dot(h, emb_out_t) + emb_out_b
    pool = jnp.sum(h, axis=0, keepdims=True) * (1.0 / _N)
    z = jnp.maximum(jnp.dot(pool, m0_t) + m0_b, 0.0)
    out_ref[...] = (jnp.dot(z, m1_t) + m1_b).reshape(1, 1, _OUT_NF)


def _blockdiag2(wt):
    z = jnp.zeros_like(wt)
    return jnp.concatenate([jnp.concatenate([wt, z], axis=1),
                            jnp.concatenate([z, wt], axis=1)], axis=0)


def kernel(h, x, params):
    p = params
    zh = jnp.zeros((1, _HID), dtype=jnp.float32)
    z1 = jnp.zeros((_HID, 1), dtype=jnp.float32)
    ws = [p["emb_in"]["W"].T, p["emb_in"]["b"].reshape(1, _HID)]
    for lp in p["layers"]:
        we0 = lp["edge0"]["W"]
        wrad = we0[:, 2 * _HID].reshape(1, _HID)
        const = (lp["edge0"]["b"] + we0[:, 2 * _HID + 1]).reshape(1, _HID)
        we1_t = lp["edge1"]["W"].T
        be1 = lp["edge1"]["b"].reshape(1, _HID)
        wc1_t = lp["coord1"]["W"].T
        ws += [
            we0[:, :_HID].T,
            we0[:, _HID:2 * _HID].T,
            const,
            jnp.concatenate([wrad, zh], axis=1),
            jnp.concatenate([zh, wrad], axis=1),
            jnp.concatenate([const, const], axis=1),
            we1_t, be1,
            _blockdiag2(we1_t),
            jnp.concatenate([be1, be1], axis=1),
            _blockdiag2(lp["coord0"]["W"].T),
            jnp.concatenate([lp["coord0"]["b"].reshape(1, _HID)] * 2, axis=1),
            jnp.concatenate(
                [jnp.tile(jnp.concatenate([wc1_t, z1], axis=0), (1, _HID)),
                 jnp.tile(jnp.concatenate([z1, wc1_t], axis=0), (1, _HID))],
                axis=1),
            lp["node0"]["W"][:, :_HID].T,
            lp["node0"]["W"][:, _HID:].T,
            lp["node0"]["b"].reshape(1, _HID),
            lp["node1"]["W"].T, lp["node1"]["b"].reshape(1, _HID),
        ]
    ws += [p["emb_out"]["W"].T, p["emb_out"]["b"].reshape(1, _HID),
           p["mlp0"]["W"].T, p["mlp0"]["b"].reshape(1, _HID),
           p["mlp1"]["W"].T, p["mlp1"]["b"].reshape(1, _OUT_NF)]

    w_specs = [pl.BlockSpec(a.shape, lambda b: (0,) * a.ndim) for a in ws]
    out = pl.pallas_call(
        _fwd,
        grid=(_B,),
        in_specs=[pl.BlockSpec((1, _N, _IN_NF), lambda b: (b, 0, 0)),
                  pl.BlockSpec((1, _N, 3), lambda b: (b, 0, 0))] + w_specs,
        out_specs=pl.BlockSpec((1, 1, _OUT_NF), lambda b: (b, 0, 0)),
        out_shape=jax.ShapeDtypeStruct((_B, 1, _OUT_NF), jnp.float32),
        compiler_params=pltpu.CompilerParams(
            dimension_semantics=("parallel",)),
    )(h, x, *ws)
    return out.reshape(_B, _OUT_NF)


# speedup vs baseline: 1.7056x; 1.0140x over previous
import jax
import jax.numpy as jnp
from jax.experimental import pallas as pl
from jax.experimental.pallas import tpu as pltpu

_IN_NF = 32
_HID = 64
_OUT_NF = 16
_N_LAYERS = 2
_B = 8
_N = 256
_N2 = _N // 2
_T = 64


def _silu(v):
    h = 0.5 * v
    return h * jnp.tanh(h) + h


def _fwd(h_ref, x_ref, *rest):
    out_ref = rest[-1]
    w = [r[...] for r in rest[:-1]]
    it = iter(w)

    emb_in_t, emb_in_b = next(it), next(it)
    layers = []
    for _ in range(_N_LAYERS):
        layers.append(dict(
            wr_t=next(it), wc_t=next(it), const=next(it),
            wradl=next(it), wradr=next(it), constp=next(it),
            we1_t=next(it), be1=next(it), we1_p=next(it), be1_p=next(it),
            wc0_p=next(it), bc0_p=next(it), wrep=next(it),
            wn0h_t=next(it), wn0a_t=next(it), bn0=next(it),
            wn1_t=next(it), bn1=next(it),
        ))
    emb_out_t, emb_out_b = next(it), next(it)
    m0_t, m0_b, m1_t, m1_b = next(it), next(it), next(it), next(it)

    h = jnp.dot(h_ref[0], emb_in_t) + emb_in_b
    x = x_ref[0]

    for lp in layers:
        hr = jnp.dot(h, lp["wr_t"])
        hc = jnp.dot(h, lp["wc_t"])

        ef_d = _silu(jnp.dot(_silu(hr + hc + lp["const"]), lp["we1_t"])
                     + lp["be1"])

        bf = jnp.bfloat16
        hrp = jnp.concatenate([hr, hr], axis=1).astype(bf)
        hcp = (jnp.concatenate([hc[:_N2], hc[_N2:]], axis=1)
               + lp["constp"]).astype(bf)
        wradl3 = lp["wradl"].reshape(1, 1, 2 * _HID).astype(bf)
        wradr3 = lp["wradr"].reshape(1, 1, 2 * _HID).astype(bf)
        we1_pb = lp["we1_p"].astype(bf)
        be1_pb = lp["be1_p"].astype(bf)
        wc0_pb = lp["wc0_p"].astype(bf)
        bc0_pb = lp["bc0_p"].astype(bf)
        wrep_b = lp["wrep"].astype(bf)

        xx = jnp.sum(x * x, axis=1)
        xxr = xx.reshape(1, _N)
        gram = jax.lax.dot_general(x, x, (((1,), (1,)), ((), ())))
        x1, x2 = x[:_N2], x[_N2:]

        on1 = jnp.ones((_N2, 1), dtype=jnp.float32)
        zpad = jnp.zeros((_N2, _HID - 4), dtype=jnp.float32)
        uaug = jnp.concatenate([x1, on1, zpad, x2, on1, zpad],
                               axis=1).reshape(1, _N2, 2 * _HID)

        agg_tiles, num_tiles = [], []
        for t in range(_N // _T):
            sl = slice(t * _T, (t + 1) * _T)
            radial = xx[sl].reshape(_T, 1) + xxr - 2.0 * gram[sl]
            rad1 = radial[:, :_N2].astype(bf)
            rad2 = radial[:, _N2:].astype(bf)
            t0 = ((rad1[:, :, None] * wradl3 + hrp[sl][:, None, :])
                  + (rad2[:, :, None] * wradr3 + hcp[None, :, :]))
            s1 = _silu(t0).reshape(_T * _N2, 2 * _HID)
            z1 = jnp.dot(s1, we1_pb, preferred_element_type=jnp.float32)
            ef = _silu(z1.astype(bf) + be1_pb)
            z2 = jnp.dot(ef, wc0_pb, preferred_element_type=jnp.float32)
            c0 = _silu(z2.astype(bf) + bc0_pb)
            srep = jnp.dot(c0, wrep_b, preferred_element_type=jnp.float32)
            p3 = srep.reshape(_T, _N2, 2 * _HID) * uaug
            r = jnp.sum(p3, axis=1)

            aggp = jnp.sum(ef.reshape(_T, _N2, 2 * _HID), axis=1,
                           dtype=jnp.float32)
            agg_tiles.append(aggp[:, :_HID] + aggp[:, _HID:])
            srow = r[:, 3:4] + r[:, _HID + 3:_HID + 4]
            sx = r[:, 0:3] + r[:, _HID:_HID + 3]
            num_tiles.append(x[sl] * srow - sx)

        agg = jnp.concatenate(agg_tiles, axis=0) - ef_d
        num = jnp.concatenate(num_tiles, axis=0)
        x = x + num * (1.0 / (_N - 1))

        hn = _silu(jnp.dot(h, lp["wn0h_t"]) + jnp.dot(agg, lp["wn0a_t"])
                   + lp["bn0"])
        h = h + jnp.dot(hn, lp["wn1_t"]) + lp["bn1"]

    h = jnp.dot(h, emb_out_t) + emb_out_b
    pool = jnp.sum(h, axis=0, keepdims=True) * (1.0 / _N)
    z = jnp.maximum(jnp.dot(pool, m0_t) + m0_b, 0.0)
    out_ref[...] = (jnp.dot(z, m1_t) + m1_b).reshape(1, 1, _OUT_NF)


def _blockdiag2(wt):
    z = jnp.zeros_like(wt)
    return jnp.concatenate([jnp.concatenate([wt, z], axis=1),
                            jnp.concatenate([z, wt], axis=1)], axis=0)


def kernel(h, x, params):
    p = params
    zh = jnp.zeros((1, _HID), dtype=jnp.float32)
    z1 = jnp.zeros((_HID, 1), dtype=jnp.float32)
    ws = [p["emb_in"]["W"].T, p["emb_in"]["b"].reshape(1, _HID)]
    for lp in p["layers"]:
        we0 = lp["edge0"]["W"]
        wrad = we0[:, 2 * _HID].reshape(1, _HID)
        const = (lp["edge0"]["b"] + we0[:, 2 * _HID + 1]).reshape(1, _HID)
        we1_t = lp["edge1"]["W"].T
        be1 = lp["edge1"]["b"].reshape(1, _HID)
        wc1_t = lp["coord1"]["W"].T
        ws += [
            we0[:, :_HID].T,
            we0[:, _HID:2 * _HID].T,
            const,
            jnp.concatenate([wrad, zh], axis=1),
            jnp.concatenate([zh, wrad], axis=1),
            jnp.concatenate([const, const], axis=1),
            we1_t, be1,
            _blockdiag2(we1_t),
            jnp.concatenate([be1, be1], axis=1),
            _blockdiag2(lp["coord0"]["W"].T),
            jnp.concatenate([lp["coord0"]["b"].reshape(1, _HID)] * 2, axis=1),
            jnp.concatenate(
                [jnp.tile(jnp.concatenate([wc1_t, z1], axis=0), (1, _HID)),
                 jnp.tile(jnp.concatenate([z1, wc1_t], axis=0), (1, _HID))],
                axis=1),
            lp["node0"]["W"][:, :_HID].T,
            lp["node0"]["W"][:, _HID:].T,
            lp["node0"]["b"].reshape(1, _HID),
            lp["node1"]["W"].T, lp["node1"]["b"].reshape(1, _HID),
        ]
    ws += [p["emb_out"]["W"].T, p["emb_out"]["b"].reshape(1, _HID),
           p["mlp0"]["W"].T, p["mlp0"]["b"].reshape(1, _HID),
           p["mlp1"]["W"].T, p["mlp1"]["b"].reshape(1, _OUT_NF)]

    w_specs = [pl.BlockSpec(a.shape, lambda b: (0,) * a.ndim) for a in ws]
    out = pl.pallas_call(
        _fwd,
        grid=(_B,),
        in_specs=[pl.BlockSpec((1, _N, _IN_NF), lambda b: (b, 0, 0)),
                  pl.BlockSpec((1, _N, 3), lambda b: (b, 0, 0))] + w_specs,
        out_specs=pl.BlockSpec((1, 1, _OUT_NF), lambda b: (b, 0, 0)),
        out_shape=jax.ShapeDtypeStruct((_B, 1, _OUT_NF), jnp.float32),
        compiler_params=pltpu.CompilerParams(
            dimension_semantics=("parallel",)),
    )(h, x, *ws)
    return out.reshape(_B, _OUT_NF)
